# Initial kernel scaffold; baseline (speedup 1.0000x reference)
import jax
import jax.numpy as jnp
from jax.experimental import pallas as pl


def kernel(feat, edge_index, W1):
    raise NotImplementedError("write your pallas kernel here")


# speedup vs baseline: 1.6359x; 1.6359x over previous
import functools

import jax
import jax.numpy as jnp
from jax import lax
from jax.experimental import pallas as pl
from jax.experimental.pallas import tpu as pltpu
from jax.experimental.pallas import tpu_sc as plsc

NC = 2
NS = 16
NW = NC * NS
L = 16


def _mm_body(f_ref, w_ref, o_ref):
    o_ref[...] = lax.dot_general(
        f_ref[...], w_ref[...],
        dimension_numbers=(((1,), (1,)), ((), ())),
        preferred_element_type=jnp.float32,
    )


def _matmul(feat, w1):
    n, d_in = feat.shape
    d_out = w1.shape[0]
    blk = 2000
    grid = n // blk
    return pl.pallas_call(
        _mm_body,
        grid=(grid,),
        in_specs=[
            pl.BlockSpec((blk, d_in), lambda i: (i, 0)),
            pl.BlockSpec((d_out, d_in), lambda i: (0, 0)),
        ],
        out_specs=pl.BlockSpec((blk, d_out), lambda i: (i, 0)),
        out_shape=jax.ShapeDtypeStruct((n, d_out), jnp.float32),
    )(feat, w1)


def _seg_max_sc(norm_h, edge_index, n_nodes, n_edges, d):
    R = (n_nodes + NW - 1) // NW
    R = ((R + 7) // 8) * 8
    n_pad = R * NW
    C = 8000
    n_chunks = n_edges // C
    assert n_chunks * C == n_edges and C % L == 0
    DC = d // L

    mesh = plsc.VectorSubcoreMesh(core_axis_name="c", subcore_axis_name="s")

    @functools.partial(
        pl.kernel,
        out_type=jax.ShapeDtypeStruct((n_pad, d), jnp.float32),
        mesh=mesh,
        compiler_params=pltpu.CompilerParams(needs_layout_passes=False),
        scratch_types=[
            pltpu.VMEM((R + 1, d), jnp.float32),
            pltpu.VMEM((C,), jnp.int32),
            pltpu.VMEM((C,), jnp.int32),
            pltpu.VMEM((C + L,), jnp.int32),
            pltpu.VMEM((C + L,), jnp.int32),
            pltpu.VMEM((L, d), jnp.float32),
            pltpu.SemaphoreType.DMA,
        ],
    )
    def seg_max(h_hbm, esrc_hbm, edst_hbm, out_hbm, acc, src_v, dst_v, srcl,
                dstl, rows, sem):
        wid = lax.axis_index("s") * NC + lax.axis_index("c")
        lo = wid * R

        neg_inf = jnp.full((L,), -jnp.inf, dtype=jnp.float32)

        def init_row(r, _):
            for c in range(DC):
                acc[r, pl.ds(c * L, L)] = neg_inf
            return 0

        lax.fori_loop(0, R + 1, init_row, 0)

        def do_chunk(k, _):
            off = pl.multiple_of(k * C, 8)
            pltpu.sync_copy(esrc_hbm.at[pl.ds(off, C)], src_v)
            pltpu.sync_copy(edst_hbm.at[pl.ds(off, C)], dst_v)

            def filt(i, n):
                s = src_v[pl.ds(i * L, L)]
                t = dst_v[pl.ds(i * L, L)] - lo
                m = (t >= 0) & (t < R)
                cs = plsc.cumsum(m.astype(jnp.int32))
                pos = n + cs - 1
                plsc.store_scatter(srcl, [pos], s, mask=m)
                plsc.store_scatter(dstl, [pos], t, mask=m)
                return n + cs[L - 1]

            n = lax.fori_loop(0, C // L, filt, 0)

            srcl[pl.ds(n, L)] = jnp.zeros((L,), jnp.int32)
            dstl[pl.ds(n, L)] = jnp.full((L,), R, jnp.int32)

            def group(g, _):
                j0 = pl.multiple_of(g * L, 8)
                pltpu.async_copy(h_hbm.at[srcl.at[pl.ds(j0, L)]], rows,
                                 sem).wait()
                dv = dstl[pl.ds(j0, L)]
                for e in range(L):
                    t = dv[e]
                    for c in range(DC):
                        sl = pl.ds(c * L, L)
                        acc[t, sl] = jnp.maximum(acc[t, sl], rows[e, sl])
                return 0

            lax.fori_loop(0, (n + L - 1) // L, group, 0)
            return 0

        lax.fori_loop(0, n_chunks, do_chunk, 0)

        def fix_row(r, _):
            for c in range(DC):
                v = acc[r, pl.ds(c * L, L)]
                acc[r, pl.ds(c * L, L)] = jnp.where(v == -jnp.inf, 0.0, v)
            return 0

        lax.fori_loop(0, R, fix_row, 0)
        pltpu.sync_copy(acc.at[pl.ds(0, R)], out_hbm.at[pl.ds(lo, R)])

    return seg_max(norm_h, edge_index[0], edge_index[1])


@jax.jit
def kernel(feat, edge_index, W1):
    n_nodes, d_in = feat.shape
    d_out = W1.shape[0]
    n_edges = edge_index.shape[1]
    norm_h = _matmul(feat, W1)
    h_pad = _seg_max_sc(norm_h, edge_index, n_nodes, n_edges, d_out)
    return jnp.concatenate([feat, h_pad[:n_nodes]], axis=1)


# speedup vs baseline: 1.9949x; 1.2195x over previous
import functools

import jax
import jax.numpy as jnp
from jax import lax
from jax.experimental import pallas as pl
from jax.experimental.pallas import tpu as pltpu
from jax.experimental.pallas import tpu_sc as plsc

NC = 2
NS = 16
NW = NC * NS
L = 16


def _mm_body(f_ref, w_ref, o_ref):
    o_ref[...] = lax.dot_general(
        f_ref[...], w_ref[...],
        dimension_numbers=(((1,), (1,)), ((), ())),
        preferred_element_type=jnp.float32,
    )


def _matmul(feat, w1):
    n, d_in = feat.shape
    d_out = w1.shape[0]
    blk = 2000
    grid = n // blk
    return pl.pallas_call(
        _mm_body,
        grid=(grid,),
        in_specs=[
            pl.BlockSpec((blk, d_in), lambda i: (i, 0)),
            pl.BlockSpec((d_out, d_in), lambda i: (0, 0)),
        ],
        out_specs=pl.BlockSpec((blk, d_out), lambda i: (i, 0)),
        out_shape=jax.ShapeDtypeStruct((n, d_out), jnp.float32),
    )(feat, w1)


def _seg_max_sc(norm_h, edge_index, n_nodes, n_edges, d):
    R = (n_nodes + NW - 1) // NW
    R = ((R + 7) // 8) * 8
    n_pad = R * NW
    C = 8000
    G = 32
    n_chunks = n_edges // C
    assert n_chunks * C == n_edges and C % L == 0
    DC = d // L

    mesh = plsc.VectorSubcoreMesh(core_axis_name="c", subcore_axis_name="s")

    @functools.partial(
        pl.kernel,
        out_type=jax.ShapeDtypeStruct((n_pad, d), jnp.float32),
        mesh=mesh,
        compiler_params=pltpu.CompilerParams(needs_layout_passes=False),
        scratch_types=[
            pltpu.VMEM((R + 1, d), jnp.float32),
            pltpu.VMEM((C,), jnp.int32),
            pltpu.VMEM((C,), jnp.int32),
            pltpu.VMEM((C + 2 * G,), jnp.int32),
            pltpu.VMEM((C + 2 * G,), jnp.int32),
            pltpu.VMEM((2, G, d), jnp.float32),
            pltpu.SemaphoreType.DMA,
        ],
    )
    def seg_max(h_hbm, esrc_hbm, edst_hbm, out_hbm, acc, src_v, dst_v, srcl,
                dstl, rows, sem):
        wid = lax.axis_index("s") * NC + lax.axis_index("c")
        lo = wid * R

        neg_inf = jnp.full((L,), -jnp.inf, dtype=jnp.float32)

        def init_row(r, _):
            for c in range(DC):
                acc[r, pl.ds(c * L, L)] = neg_inf
            return 0

        lax.fori_loop(0, R + 1, init_row, 0)

        def do_chunk(k, _):
            off = pl.multiple_of(k * C, 8)
            pltpu.sync_copy(esrc_hbm.at[pl.ds(off, C)], src_v)
            pltpu.sync_copy(edst_hbm.at[pl.ds(off, C)], dst_v)

            def filt(i, n):
                s = src_v[pl.ds(i * L, L)]
                t = dst_v[pl.ds(i * L, L)] - lo
                m = (t >= 0) & (t < R)
                cs = plsc.cumsum(m.astype(jnp.int32))
                pos = n + cs - 1
                plsc.store_scatter(srcl, [pos], s, mask=m)
                plsc.store_scatter(dstl, [pos], t, mask=m)
                return n + cs[L - 1]

            n = lax.fori_loop(0, C // L, filt, 0)

            for p in range(G // L):
                srcl[pl.ds(n + p * L, L)] = jnp.zeros((L,), jnp.int32)
                dstl[pl.ds(n + p * L, L)] = jnp.full((L,), R, jnp.int32)

            nsg = (n + G - 1) // G

            def issue(g):
                j0 = pl.multiple_of(g * G, 8)
                return pltpu.async_copy(
                    h_hbm.at[srcl.at[pl.ds(j0, G)]], rows.at[g & 1], sem)

            @pl.when(nsg > 0)
            def _prime():
                issue(0)

            def group(g, _):
                @pl.when(g + 1 < nsg)
                def _ahead():
                    issue(g + 1)

                b = g & 1
                j0 = pl.multiple_of(g * G, 8)
                pltpu.make_async_copy(
                    h_hbm.at[srcl.at[pl.ds(j0, G)]], rows.at[b], sem).wait()
                for q in range(G // L):
                    dv = dstl[pl.ds(j0 + q * L, L)]
                    for e in range(L):
                        t = dv[e]
                        for c in range(DC):
                            sl = pl.ds(c * L, L)
                            acc[t, sl] = jnp.maximum(acc[t, sl],
                                                     rows[b, q * L + e, sl])
                return 0

            lax.fori_loop(0, nsg, group, 0)
            return 0

        lax.fori_loop(0, n_chunks, do_chunk, 0)

        def fix_row(r, _):
            for c in range(DC):
                v = acc[r, pl.ds(c * L, L)]
                acc[r, pl.ds(c * L, L)] = jnp.where(v == -jnp.inf, 0.0, v)
            return 0

        lax.fori_loop(0, R, fix_row, 0)
        pltpu.sync_copy(acc.at[pl.ds(0, R)], out_hbm.at[pl.ds(lo, R)])

    return seg_max(norm_h, edge_index[0], edge_index[1])


@jax.jit
def kernel(feat, edge_index, W1):
    n_nodes, d_in = feat.shape
    d_out = W1.shape[0]
    n_edges = edge_index.shape[1]
    norm_h = _matmul(feat, W1)
    h_pad = _seg_max_sc(norm_h, edge_index, n_nodes, n_edges, d_out)
    return jnp.concatenate([feat, h_pad[:n_nodes]], axis=1)
